# baseline (speedup 1.0000x reference)
import functools

import jax
import jax.numpy as jnp
from jax import lax
from jax.experimental import pallas as pl
from jax.experimental.pallas import tpu as pltpu
from jax.experimental.pallas import tpu_sc as plsc

N = 10000
E = 320000
D = 128
R = 8
RN = R * N

NC = 2
NS = 16
NW = NC * NS
EW = E // NW
K = 80
NCH = EW // K
RPT = 624
ZB = 208
CHK = 4000

_f32 = jnp.float32
_i32 = jnp.int32


def _mesh():
    return plsc.VectorSubcoreMesh(
        core_axis_name="c", subcore_axis_name="s", num_cores=NC, num_subcores=NS
    )


def _hist_body(cidx_hbm, zeros_hbm, out_hbm, cidx_v, ones_v, zer_v, hist_sh):
    c = lax.axis_index("c")
    s = lax.axis_index("s")
    wid = s * NC + c
    ones16 = jnp.ones((16,), _f32)
    for q in range(K // 16):
        ones_v[pl.ds(q * 16, 16)] = ones16
    pltpu.sync_copy(zeros_hbm, zer_v)
    pltpu.sync_copy(zer_v, hist_sh.at[pl.ds(s * (RN // NS), RN // NS)])
    plsc.subcore_barrier()
    pltpu.sync_copy(cidx_hbm.at[wid], cidx_v)

    def chunk(j, carry):
        pltpu.sync_copy(ones_v, hist_sh.at[cidx_v.at[j]], add=True)
        return carry

    lax.fori_loop(0, NCH, chunk, 0)
    plsc.subcore_barrier()
    pltpu.sync_copy(hist_sh.at[pl.ds(s * (RN // NS), RN // NS)], zer_v)
    pltpu.sync_copy(zer_v, out_hbm.at[pl.ds(c * RN + s * (RN // NS), RN // NS)])


def _hist_call(cidx3, zeros_h):
    return pl.kernel(
        _hist_body,
        out_type=jax.ShapeDtypeStruct((NC * RN,), _f32),
        mesh=_mesh(),
        scratch_types=[
            pltpu.VMEM((NCH, K), _i32),
            pltpu.VMEM((K,), _f32),
            pltpu.VMEM((RN // NS,), _f32),
            pltpu.VMEM_SHARED((RN,), _f32),
        ],
    )(cidx3, zeros_h)


EW2 = E // NS
NCH2 = EW2 // K


def _scale_body(part_hbm, cidx_hbm, out_hbm, table_v, p0_v, p1_v, cidx_v, out_v):
    c = lax.axis_index("c")
    s = lax.axis_index("s")
    wid = s * NC + c
    one = jnp.ones((16,), _f32)
    for t in range(RN // CHK):
        pltpu.sync_copy(part_hbm.at[pl.ds(t * CHK, CHK)], p0_v)
        pltpu.sync_copy(part_hbm.at[pl.ds(RN + t * CHK, CHK)], p1_v)

        def vec(i, carry, t=t):
            sl = pl.ds(i * 16, 16)
            v = p0_v[sl] + p1_v[sl]
            table_v[pl.ds(t * CHK + i * 16, 16)] = one / jnp.maximum(v, 1.0)
            return carry

        lax.fori_loop(0, CHK // 16, vec, 0)
    pltpu.sync_copy(cidx_hbm.at[wid], cidx_v)

    def chunk(j, carry):
        for k in range(K // 16):
            iv = cidx_v[j, pl.ds(k * 16, 16)]
            out_v[j, pl.ds(k * 16, 16)] = plsc.load_gather(table_v, [iv])
        return carry

    lax.fori_loop(0, NCH, chunk, 0)
    pltpu.sync_copy(out_v, out_hbm.at[wid])


def _scale_call(partials, cidx3):
    return pl.kernel(
        _scale_body,
        out_type=jax.ShapeDtypeStruct((NW, NCH, K), _f32),
        mesh=_mesh(),
        compiler_params=pltpu.CompilerParams(needs_layout_passes=False),
        scratch_types=[
            pltpu.VMEM((RN,), _f32),
            pltpu.VMEM((CHK,), _f32),
            pltpu.VMEM((CHK,), _f32),
            pltpu.VMEM((NCH, K), _i32),
            pltpu.VMEM((NCH, K), _f32),
        ],
    )(partials, cidx3)


EW2 = E // NS
NCH2 = EW2 // K
DH = D // NC


def _msg_body(y_hbm, gidx_hbm, dst_hbm, scale_hbm, zeros_hbm, out_hbm,
              gidx_v, dst_v, scale_v, rows0_v, rows1_v, zbuf_v, acc_sh,
              sg0, sg1, ss0, ss1):
    c = lax.axis_index("c")
    s = lax.axis_index("s")
    base = s * RPT
    pltpu.sync_copy(zeros_hbm, zbuf_v)
    for t in range(RPT // ZB):
        pltpu.sync_copy(zbuf_v, acc_sh.at[pl.ds(base + t * ZB, ZB)])

    @pl.when(s == NS - 1)
    def _zero_tail():
        pltpu.sync_copy(zbuf_v.at[pl.ds(0, 16)],
                        acc_sh.at[pl.ds(NS * RPT, N - NS * RPT)])

    plsc.subcore_barrier()
    pltpu.sync_copy(gidx_hbm.at[s], gidx_v)
    pltpu.sync_copy(dst_hbm.at[s], dst_v)
    pltpu.sync_copy(scale_hbm.at[s], scale_v)

    par16 = lax.iota(_i32, 16) % 2

    def fix(j, carry):
        for k in range(K // 16):
            sl = pl.ds(k * 16, 16)
            gidx_v[j, sl] = gidx_v[j, sl] * 4 + 2 * c + par16
        return carry

    lax.fori_loop(0, NCH2, fix, 0)

    bufs = (rows0_v, rows1_v)
    sgs = (sg0, sg1)
    sss = (ss0, ss1)

    class _op:
        def __init__(self, start, wait):
            self.start = start
            self.wait = wait

    def _gather(j, b):
        return _op(
            lambda: pltpu.async_copy(y_hbm.at[gidx_v.at[j]], bufs[b], sgs[b]),
            lambda: pltpu.make_async_copy(
                y_hbm.at[gidx_v.at[j]], bufs[b], sgs[b]
            ).wait(),
        )

    def _scatter(j, b):
        return _op(
            lambda: pltpu.async_copy(
                bufs[b], acc_sh.at[dst_v.at[j]], sss[b], add=True
            ),
            lambda: pltpu.make_async_copy(
                bufs[b], acc_sh.at[dst_v.at[j]], sss[b]
            ).wait(),
        )

    def _compute(j, b):
        buf = bufs[b]
        for i16 in range(K // 16):
            svec = scale_v[pl.ds(j * K + i16 * 16, 16)]
            for l in range(16):
                sv = lax.gather(
                    svec,
                    jnp.full((16, 1), l, _i32),
                    lax.GatherDimensionNumbers(
                        offset_dims=(),
                        collapsed_slice_dims=(0,),
                        start_index_map=(0,),
                    ),
                    slice_sizes=(1,),
                    mode=lax.GatherScatterMode.PROMISE_IN_BOUNDS,
                )
                i = i16 * 16 + l
                for q in range(DH // 16):
                    sl = pl.ds(q * 16, 16)
                    buf[i, sl] = buf[i, sl] * sv

    _gather(0, 0).start()

    def pair(p, carry):
        j0 = 2 * p
        _gather(j0, 0).wait()

        @pl.when(p > 0)
        def _wait_s1():
            _scatter(j0 - 1, 1).wait()

        _gather(j0 + 1, 1).start()
        _compute(j0, 0)
        _scatter(j0, 0).start()
        _gather(j0 + 1, 1).wait()
        _compute(j0 + 1, 1)
        _scatter(j0, 0).wait()

        @pl.when(p < NCH2 // 2 - 1)
        def _next_g0():
            _gather(j0 + 2, 0).start()

        _scatter(j0 + 1, 1).start()
        return carry

    lax.fori_loop(0, NCH2 // 2, pair, 0)
    _scatter(NCH2 - 1, 1).wait()
    plsc.subcore_barrier()
    for t in range(RPT // ZB):
        r0 = base + t * ZB
        pltpu.sync_copy(acc_sh.at[pl.ds(r0, ZB)], zbuf_v)
        pltpu.sync_copy(zbuf_v, out_hbm.at[c, pl.ds(r0, ZB)])

    @pl.when(s == NS - 1)
    def _out_tail():
        pltpu.sync_copy(acc_sh.at[pl.ds(NS * RPT, N - NS * RPT)],
                        zbuf_v.at[pl.ds(0, 16)])
        pltpu.sync_copy(zbuf_v.at[pl.ds(0, 16)],
                        out_hbm.at[c, pl.ds(NS * RPT, N - NS * RPT)])


def _msg_call(y2, gidx2, dst2, scale2, zeros2):
    return pl.kernel(
        _msg_body,
        out_type=jax.ShapeDtypeStruct((NC, N, DH), _f32),
        mesh=_mesh(),
        compiler_params=pltpu.CompilerParams(
            needs_layout_passes=False, use_tc_tiling_on_sc=False),
        scratch_types=[
            pltpu.VMEM((NCH2, K), _i32),
            pltpu.VMEM((NCH2, K), _i32),
            pltpu.VMEM((EW2,), _f32),
            pltpu.VMEM((K, DH), _f32),
            pltpu.VMEM((K, DH), _f32),
            pltpu.VMEM((ZB, DH), _f32),
            pltpu.VMEM_SHARED((N, DH), _f32),
            pltpu.SemaphoreType.DMA,
            pltpu.SemaphoreType.DMA,
            pltpu.SemaphoreType.DMA,
            pltpu.SemaphoreType.DMA,
        ],
    )(y2, gidx2, dst2, scale2, zeros2)


BN = 1000


def _mm1_body(h_ref, w_ref, b_ref, y8_ref, aux_ref):
    g = pl.program_id(1)
    d = jnp.dot(h_ref[...], w_ref[0], preferred_element_type=_f32) + b_ref[0]

    @pl.when(g < R)
    def _wy():
        y8_ref[0] = jnp.concatenate(
            [d[:, :DH], d[:, :DH], d[:, DH:], d[:, DH:]], axis=-1
        )

    @pl.when(g >= R)
    def _wa():
        aux_ref[0] = d


def _mm1_call(h, wstack, bstack):
    g = wstack.shape[0]
    na = g - R
    return pl.pallas_call(
        _mm1_body,
        grid=(N // BN, g),
        in_specs=[
            pl.BlockSpec((BN, D), lambda i, j: (i, 0)),
            pl.BlockSpec((1, D, D), lambda i, j: (j, 0, 0)),
            pl.BlockSpec((1, 1, D), lambda i, j: (j, 0, 0)),
        ],
        out_specs=[
            pl.BlockSpec((1, BN, 2 * D), lambda i, j: (jnp.minimum(j, R - 1), i, 0)),
            pl.BlockSpec((1, BN, D), lambda i, j: (jnp.maximum(j - R, 0), i, 0)),
        ],
        out_shape=[
            jax.ShapeDtypeStruct((R, N, 2 * D), _f32),
            jax.ShapeDtypeStruct((na, N, D), _f32),
        ],
    )(h, wstack, bstack.reshape(g, 1, D))


def _mmx_body(base_ref, res_ref, p0_ref, p1_ref, w_ref, b_ref,
              y8_ref, aux_ref, h_scr):
    g = pl.program_id(1)

    @pl.when(g == 0)
    def _mkh():
        p = jnp.concatenate([p0_ref[0], p1_ref[0]], axis=-1)
        h_scr[...] = res_ref[0] + jnp.maximum(base_ref[0] + p, 0.0)

    d = jnp.dot(h_scr[...], w_ref[0], preferred_element_type=_f32) + b_ref[0]

    @pl.when(g < R)
    def _wy():
        y8_ref[0] = jnp.concatenate(
            [d[:, :DH], d[:, :DH], d[:, DH:], d[:, DH:]], axis=-1
        )

    @pl.when(g >= R)
    def _wa():
        aux_ref[0] = d


def _mmx_call(aux_base, bi, aux_res, ri, parts, wstack, bstack):
    g = wstack.shape[0]
    return pl.pallas_call(
        _mmx_body,
        grid=(N // BN, g),
        in_specs=[
            pl.BlockSpec((1, BN, D), lambda i, j, bi=bi: (bi, i, 0)),
            pl.BlockSpec((1, BN, D), lambda i, j, ri=ri: (ri, i, 0)),
            pl.BlockSpec((1, BN, DH), lambda i, j: (0, i, 0)),
            pl.BlockSpec((1, BN, DH), lambda i, j: (1, i, 0)),
            pl.BlockSpec((1, D, D), lambda i, j: (j, 0, 0)),
            pl.BlockSpec((1, 1, D), lambda i, j: (j, 0, 0)),
        ],
        out_specs=[
            pl.BlockSpec((1, BN, 2 * D), lambda i, j: (jnp.minimum(j, R - 1), i, 0)),
            pl.BlockSpec((1, BN, D), lambda i, j: (jnp.maximum(j - R, 0), i, 0)),
        ],
        out_shape=[
            jax.ShapeDtypeStruct((R, N, 2 * D), _f32),
            jax.ShapeDtypeStruct((g - R, N, D), _f32),
        ],
        scratch_shapes=[pltpu.VMEM((BN, D), _f32)],
    )(aux_base, aux_res, parts, parts, wstack, bstack.reshape(g, 1, D))


def _comb_body(base_ref, res_ref, p0_ref, p1_ref, o_ref):
    p = jnp.concatenate([p0_ref[0], p1_ref[0]], axis=-1)
    o_ref[...] = res_ref[0] + jnp.maximum(base_ref[0] + p, 0.0)


def _comb_call(aux_base, bi, aux_res, ri, parts):
    return pl.pallas_call(
        _comb_body,
        grid=(N // BN,),
        in_specs=[
            pl.BlockSpec((1, BN, D), lambda i, bi=bi: (bi, i, 0)),
            pl.BlockSpec((1, BN, D), lambda i, ri=ri: (ri, i, 0)),
            pl.BlockSpec((1, BN, DH), lambda i: (0, i, 0)),
            pl.BlockSpec((1, BN, DH), lambda i: (1, i, 0)),
        ],
        out_specs=pl.BlockSpec((BN, D), lambda i: (i, 0)),
        out_shape=jax.ShapeDtypeStruct((N, D), _f32),
    )(aux_base, aux_res, parts, parts)


def kernel(x, edge_index, edge_type, W1, root1, b1, W2, root2, b2,
           W3, root3, b3, Wres, bres):
    src = edge_index[0]
    dst = edge_index[1]
    gidx2 = (edge_type * N + src).reshape(NS, NCH2, K)
    cidx3 = (edge_type * N + dst).reshape(NW, NCH, K)
    dst2 = dst.reshape(NS, NCH2, K)
    zeros_h = jnp.zeros((RN // NS,), _f32)
    zeros2 = jnp.zeros((ZB, DH), _f32)
    zb8 = jnp.zeros((R, D), _f32)

    partials = _hist_call(cidx3, zeros_h)
    scale2 = _scale_call(partials, cidx3).reshape(NS, EW2)

    w1 = jnp.concatenate([W1, root1[None], Wres[None]], axis=0)
    b1s = jnp.concatenate([zb8, b1[None], bres[None]], axis=0)
    y8, aux1 = _mm1_call(x, w1, b1s)
    parts1 = _msg_call(y8.reshape(4 * RN, DH), gidx2, dst2, scale2, zeros2)

    w2 = jnp.concatenate([W2, root2[None]], axis=0)
    b2s = jnp.concatenate([zb8, b2[None]], axis=0)
    y8, aux2 = _mmx_call(aux1, 0, aux1, 1, parts1, w2, b2s)
    parts2 = _msg_call(y8.reshape(4 * RN, DH), gidx2, dst2, scale2, zeros2)

    w3 = jnp.concatenate([W3, root3[None]], axis=0)
    b3s = jnp.concatenate([zb8, b3[None]], axis=0)
    y8, aux3 = _mmx_call(aux2, 0, aux1, 1, parts2, w3, b3s)
    parts3 = _msg_call(y8.reshape(4 * RN, DH), gidx2, dst2, scale2, zeros2)

    return _comb_call(aux3, 0, aux1, 1, parts3)


# speedup vs baseline: 1.6546x; 1.6546x over previous
import functools

import jax
import jax.numpy as jnp
from jax import lax
from jax.experimental import pallas as pl
from jax.experimental.pallas import tpu as pltpu
from jax.experimental.pallas import tpu_sc as plsc

N = 10000
E = 320000
D = 128
R = 8
RN = R * N

NC = 2
NS = 16
NW = NC * NS
EW = E // NW
K = 80
NCH = EW // K
RPT = 624
ZB = 104
CHK = 4000

_f32 = jnp.float32
_i32 = jnp.int32


def _mesh():
    return plsc.VectorSubcoreMesh(
        core_axis_name="c", subcore_axis_name="s", num_cores=NC, num_subcores=NS
    )


def _hist_body(cidx_hbm, zeros_hbm, out_hbm, cidx_v, ones_v, zer_v, hist_sh):
    c = lax.axis_index("c")
    s = lax.axis_index("s")
    wid = s * NC + c
    ones16 = jnp.ones((16,), _f32)
    for q in range(K // 16):
        ones_v[pl.ds(q * 16, 16)] = ones16
    pltpu.sync_copy(zeros_hbm, zer_v)
    pltpu.sync_copy(zer_v, hist_sh.at[pl.ds(s * (RN // NS), RN // NS)])
    plsc.subcore_barrier()
    pltpu.sync_copy(cidx_hbm.at[wid], cidx_v)

    def chunk(j, carry):
        pltpu.sync_copy(ones_v, hist_sh.at[cidx_v.at[j]], add=True)
        return carry

    lax.fori_loop(0, NCH, chunk, 0)
    plsc.subcore_barrier()
    pltpu.sync_copy(hist_sh.at[pl.ds(s * (RN // NS), RN // NS)], zer_v)
    pltpu.sync_copy(zer_v, out_hbm.at[pl.ds(c * RN + s * (RN // NS), RN // NS)])


def _hist_call(cidx3, zeros_h):
    return pl.kernel(
        _hist_body,
        out_type=jax.ShapeDtypeStruct((NC * RN,), _f32),
        mesh=_mesh(),
        scratch_types=[
            pltpu.VMEM((NCH, K), _i32),
            pltpu.VMEM((K,), _f32),
            pltpu.VMEM((RN // NS,), _f32),
            pltpu.VMEM_SHARED((RN,), _f32),
        ],
    )(cidx3, zeros_h)


EW2 = E // NS
NCH2 = EW2 // K


def _scale_body(part_hbm, cidx_hbm, out_hbm, table_v, p0_v, p1_v, cidx_v, out_v):
    c = lax.axis_index("c")
    s = lax.axis_index("s")
    wid = s * NC + c
    one = jnp.ones((16,), _f32)
    for t in range(RN // CHK):
        pltpu.sync_copy(part_hbm.at[pl.ds(t * CHK, CHK)], p0_v)
        pltpu.sync_copy(part_hbm.at[pl.ds(RN + t * CHK, CHK)], p1_v)

        def vec(i, carry, t=t):
            sl = pl.ds(i * 16, 16)
            v = p0_v[sl] + p1_v[sl]
            table_v[pl.ds(t * CHK + i * 16, 16)] = one / jnp.maximum(v, 1.0)
            return carry

        lax.fori_loop(0, CHK // 16, vec, 0)
    pltpu.sync_copy(cidx_hbm.at[wid], cidx_v)

    def chunk(j, carry):
        for k in range(K // 16):
            iv = cidx_v[j, pl.ds(k * 16, 16)]
            out_v[j, pl.ds(k * 16, 16)] = plsc.load_gather(table_v, [iv])
        return carry

    lax.fori_loop(0, NCH, chunk, 0)
    pltpu.sync_copy(out_v, out_hbm.at[wid])


def _scale_call(partials, cidx3):
    return pl.kernel(
        _scale_body,
        out_type=jax.ShapeDtypeStruct((NW, NCH, K), _f32),
        mesh=_mesh(),
        compiler_params=pltpu.CompilerParams(needs_layout_passes=False),
        scratch_types=[
            pltpu.VMEM((RN,), _f32),
            pltpu.VMEM((CHK,), _f32),
            pltpu.VMEM((CHK,), _f32),
            pltpu.VMEM((NCH, K), _i32),
            pltpu.VMEM((NCH, K), _f32),
        ],
    )(partials, cidx3)


EW2 = E // NS
NCH2 = EW2 // K
DH = D // NC


def _msg_body(y_hbm, gidx_hbm, dst_hbm, scale_hbm, zeros_hbm, out_hbm,
              gidx_v, dst_v, scale_v, rows0_v, rows1_v, srow0_v, srow1_v,
              zbuf_v, acc_sh, sg0, sg1, ss0, ss1):
    c = lax.axis_index("c")
    s = lax.axis_index("s")
    base = s * RPT
    pltpu.sync_copy(zeros_hbm, zbuf_v)
    for t in range(RPT // ZB):
        pltpu.sync_copy(zbuf_v, acc_sh.at[pl.ds(base + t * ZB, ZB)])

    @pl.when(s == NS - 1)
    def _zero_tail():
        pltpu.sync_copy(zbuf_v.at[pl.ds(0, 16)],
                        acc_sh.at[pl.ds(NS * RPT, N - NS * RPT)])

    plsc.subcore_barrier()
    pltpu.sync_copy(gidx_hbm.at[s], gidx_v)
    pltpu.sync_copy(dst_hbm.at[s], dst_v)
    pltpu.sync_copy(scale_hbm.at[s], scale_v)

    def fix(j, carry):
        for k in range(K // 16):
            sl = pl.ds(k * 16, 16)
            gidx_v[j, sl] = gidx_v[j, sl] * 2 + c
        return carry

    lax.fori_loop(0, NCH2, fix, 0)

    bufs = (rows0_v, rows1_v)
    sbufs = (srow0_v, srow1_v)
    sgs = (sg0, sg1)
    sss = (ss0, ss1)

    class _op:
        def __init__(self, start, wait):
            self.start = start
            self.wait = wait

    def _gather(j, b):
        return _op(
            lambda: pltpu.async_copy(y_hbm.at[gidx_v.at[j]], bufs[b], sgs[b]),
            lambda: pltpu.make_async_copy(
                y_hbm.at[gidx_v.at[j]], bufs[b], sgs[b]
            ).wait(),
        )

    def _scatter(j, b):
        return _op(
            lambda: pltpu.async_copy(
                sbufs[b], acc_sh.at[dst_v.at[j]], sss[b], add=True
            ),
            lambda: pltpu.make_async_copy(
                sbufs[b], acc_sh.at[dst_v.at[j]], sss[b]
            ).wait(),
        )

    def _compute(j, b):
        buf = bufs[b]
        sbuf = sbufs[b]
        for i16 in range(K // 16):
            svec = scale_v[pl.ds(j * K + i16 * 16, 16)]
            for l in range(16):
                sv = lax.gather(
                    svec,
                    jnp.full((16, 1), l, _i32),
                    lax.GatherDimensionNumbers(
                        offset_dims=(),
                        collapsed_slice_dims=(0,),
                        start_index_map=(0,),
                    ),
                    slice_sizes=(1,),
                    mode=lax.GatherScatterMode.PROMISE_IN_BOUNDS,
                )
                i = i16 * 16 + l
                for q in range(DH // 16):
                    sl = pl.ds(q * 16, 16)
                    sbuf[i, sl] = buf[i, sl] * sv

    _gather(0, 0).start()
    _gather(1, 1).start()

    def pair(p, carry):
        j0 = 2 * p
        _gather(j0, 0).wait()

        @pl.when(p > 0)
        def _ws0():
            _scatter(j0 - 2, 0).wait()

        _compute(j0, 0)

        @pl.when(p < NCH2 // 2 - 1)
        def _g0():
            _gather(j0 + 2, 0).start()

        _scatter(j0, 0).start()
        _gather(j0 + 1, 1).wait()

        @pl.when(p > 0)
        def _ws1():
            _scatter(j0 - 1, 1).wait()

        _compute(j0 + 1, 1)

        @pl.when(p < NCH2 // 2 - 1)
        def _g1():
            _gather(j0 + 3, 1).start()

        _scatter(j0 + 1, 1).start()
        return carry

    lax.fori_loop(0, NCH2 // 2, pair, 0)
    _scatter(NCH2 - 2, 0).wait()
    _scatter(NCH2 - 1, 1).wait()
    plsc.subcore_barrier()
    for t in range(RPT // ZB):
        r0 = base + t * ZB
        pltpu.sync_copy(acc_sh.at[pl.ds(r0, ZB)], zbuf_v)
        pltpu.sync_copy(zbuf_v, out_hbm.at[c, pl.ds(r0, ZB)])

    @pl.when(s == NS - 1)
    def _out_tail():
        pltpu.sync_copy(acc_sh.at[pl.ds(NS * RPT, N - NS * RPT)],
                        zbuf_v.at[pl.ds(0, 16)])
        pltpu.sync_copy(zbuf_v.at[pl.ds(0, 16)],
                        out_hbm.at[c, pl.ds(NS * RPT, N - NS * RPT)])


def _msg_call(y2, gidx2, dst2, scale2, zeros2):
    return pl.kernel(
        _msg_body,
        out_type=jax.ShapeDtypeStruct((NC, N, DH), _f32),
        mesh=_mesh(),
        compiler_params=pltpu.CompilerParams(
            needs_layout_passes=False, use_tc_tiling_on_sc=False),
        scratch_types=[
            pltpu.VMEM((NCH2, K), _i32),
            pltpu.VMEM((NCH2, K), _i32),
            pltpu.VMEM((EW2,), _f32),
            pltpu.VMEM((K, DH), _f32),
            pltpu.VMEM((K, DH), _f32),
            pltpu.VMEM((K, DH), _f32),
            pltpu.VMEM((K, DH), _f32),
            pltpu.VMEM((ZB, DH), _f32),
            pltpu.VMEM_SHARED((N, DH), _f32),
            pltpu.SemaphoreType.DMA,
            pltpu.SemaphoreType.DMA,
            pltpu.SemaphoreType.DMA,
            pltpu.SemaphoreType.DMA,
        ],
    )(y2, gidx2, dst2, scale2, zeros2)


BN = 1000


def _mm1_body(h_ref, w_ref, b_ref, y8_ref, aux_ref):
    g = pl.program_id(1)
    d = jnp.dot(h_ref[...], w_ref[0], preferred_element_type=_f32) + b_ref[0]

    @pl.when(g < R)
    def _wy():
        y8_ref[0] = d

    @pl.when(g >= R)
    def _wa():
        aux_ref[0] = d


def _mm1_call(h, wstack, bstack):
    g = wstack.shape[0]
    na = g - R
    return pl.pallas_call(
        _mm1_body,
        grid=(N // BN, g),
        in_specs=[
            pl.BlockSpec((BN, D), lambda i, j: (i, 0)),
            pl.BlockSpec((1, D, D), lambda i, j: (j, 0, 0)),
            pl.BlockSpec((1, 1, D), lambda i, j: (j, 0, 0)),
        ],
        out_specs=[
            pl.BlockSpec((1, BN, D), lambda i, j: (jnp.minimum(j, R - 1), i, 0)),
            pl.BlockSpec((1, BN, D), lambda i, j: (jnp.maximum(j - R, 0), i, 0)),
        ],
        out_shape=[
            jax.ShapeDtypeStruct((R, N, D), _f32),
            jax.ShapeDtypeStruct((na, N, D), _f32),
        ],
    )(h, wstack, bstack.reshape(g, 1, D))


def _mmx_body(base_ref, res_ref, p0_ref, p1_ref, w_ref, b_ref,
              y8_ref, aux_ref, h_scr):
    g = pl.program_id(1)

    @pl.when(g == 0)
    def _mkh():
        p = jnp.concatenate([p0_ref[0], p1_ref[0]], axis=-1)
        h_scr[...] = res_ref[0] + jnp.maximum(base_ref[0] + p, 0.0)

    d = jnp.dot(h_scr[...], w_ref[0], preferred_element_type=_f32) + b_ref[0]

    @pl.when(g < R)
    def _wy():
        y8_ref[0] = d

    @pl.when(g >= R)
    def _wa():
        aux_ref[0] = d


def _mmx_call(aux_base, bi, aux_res, ri, parts, wstack, bstack):
    g = wstack.shape[0]
    return pl.pallas_call(
        _mmx_body,
        grid=(N // BN, g),
        in_specs=[
            pl.BlockSpec((1, BN, D), lambda i, j, bi=bi: (bi, i, 0)),
            pl.BlockSpec((1, BN, D), lambda i, j, ri=ri: (ri, i, 0)),
            pl.BlockSpec((1, BN, DH), lambda i, j: (0, i, 0)),
            pl.BlockSpec((1, BN, DH), lambda i, j: (1, i, 0)),
            pl.BlockSpec((1, D, D), lambda i, j: (j, 0, 0)),
            pl.BlockSpec((1, 1, D), lambda i, j: (j, 0, 0)),
        ],
        out_specs=[
            pl.BlockSpec((1, BN, D), lambda i, j: (jnp.minimum(j, R - 1), i, 0)),
            pl.BlockSpec((1, BN, D), lambda i, j: (jnp.maximum(j - R, 0), i, 0)),
        ],
        out_shape=[
            jax.ShapeDtypeStruct((R, N, D), _f32),
            jax.ShapeDtypeStruct((g - R, N, D), _f32),
        ],
        scratch_shapes=[pltpu.VMEM((BN, D), _f32)],
    )(aux_base, aux_res, parts, parts, wstack, bstack.reshape(g, 1, D))


def _comb_body(base_ref, res_ref, p0_ref, p1_ref, o_ref):
    p = jnp.concatenate([p0_ref[0], p1_ref[0]], axis=-1)
    o_ref[...] = res_ref[0] + jnp.maximum(base_ref[0] + p, 0.0)


def _comb_call(aux_base, bi, aux_res, ri, parts):
    return pl.pallas_call(
        _comb_body,
        grid=(N // BN,),
        in_specs=[
            pl.BlockSpec((1, BN, D), lambda i, bi=bi: (bi, i, 0)),
            pl.BlockSpec((1, BN, D), lambda i, ri=ri: (ri, i, 0)),
            pl.BlockSpec((1, BN, DH), lambda i: (0, i, 0)),
            pl.BlockSpec((1, BN, DH), lambda i: (1, i, 0)),
        ],
        out_specs=pl.BlockSpec((BN, D), lambda i: (i, 0)),
        out_shape=jax.ShapeDtypeStruct((N, D), _f32),
    )(aux_base, aux_res, parts, parts)


def kernel(x, edge_index, edge_type, W1, root1, b1, W2, root2, b2,
           W3, root3, b3, Wres, bres):
    src = edge_index[0]
    dst = edge_index[1]
    gidx2 = (edge_type * N + src).reshape(NS, NCH2, K)
    cidx3 = (edge_type * N + dst).reshape(NW, NCH, K)
    dst2 = dst.reshape(NS, NCH2, K)
    zeros_h = jnp.zeros((RN // NS,), _f32)
    zeros2 = jnp.zeros((ZB, DH), _f32)
    zb8 = jnp.zeros((R, D), _f32)

    partials = _hist_call(cidx3, zeros_h)
    scale2 = _scale_call(partials, cidx3).reshape(NS, EW2)

    w1 = jnp.concatenate([W1, root1[None], Wres[None]], axis=0)
    b1s = jnp.concatenate([zb8, b1[None], bres[None]], axis=0)
    y8, aux1 = _mm1_call(x, w1, b1s)
    parts1 = _msg_call(y8.reshape(2 * RN, DH), gidx2, dst2, scale2, zeros2)

    w2 = jnp.concatenate([W2, root2[None]], axis=0)
    b2s = jnp.concatenate([zb8, b2[None]], axis=0)
    y8, aux2 = _mmx_call(aux1, 0, aux1, 1, parts1, w2, b2s)
    parts2 = _msg_call(y8.reshape(2 * RN, DH), gidx2, dst2, scale2, zeros2)

    w3 = jnp.concatenate([W3, root3[None]], axis=0)
    b3s = jnp.concatenate([zb8, b3[None]], axis=0)
    y8, aux3 = _mmx_call(aux2, 0, aux1, 1, parts2, w3, b3s)
    parts3 = _msg_call(y8.reshape(2 * RN, DH), gidx2, dst2, scale2, zeros2)

    return _comb_call(aux3, 0, aux1, 1, parts3)
